# baseline (speedup 1.0000x reference)
import jax
import jax.numpy as jnp
from jax import lax
from jax.experimental import pallas as pl
from jax.experimental.pallas import tpu as pltpu

_F32 = jnp.float32
_N_CHUNKS = 4


def _fused_kernel(feat_ref, adj_hbm_ref, mask_ref,
                  e1w_ref, e1b_ref, e2w_ref, e2b_ref,
                  sgw_ref, sgb_ref, gdw_ref, gdb_ref,
                  p1w_ref, p1b_ref, p2w_ref, p2b_ref,
                  vw_ref, vb_ref, out_ref, adj_vmem, sems):
    n = adj_hbm_ref.shape[0]
    chunk = n // _N_CHUNKS

    copies = []
    for k in range(_N_CHUNKS):
        rows = pl.ds(k * chunk, chunk)
        cp = pltpu.make_async_copy(adj_hbm_ref.at[rows, :],
                                   adj_vmem.at[rows, :], sems.at[k])
        cp.start()
        copies.append(cp)

    x = jnp.maximum(
        jnp.dot(feat_ref[...], e1w_ref[...], preferred_element_type=_F32)
        + e1b_ref[...], 0.0)
    x = jnp.maximum(
        jnp.dot(x, e2w_ref[...], preferred_element_type=_F32)
        + e2b_ref[...], 0.0)

    deg = jnp.zeros((1, n), dtype=_F32)
    for k in range(_N_CHUNKS):
        copies[k].wait()
        deg = deg + jnp.sum(adj_vmem[pl.ds(k * chunk, chunk), :], axis=0,
                            keepdims=True)
    d_row = jnp.where(deg > 0.0, lax.rsqrt(deg), 0.0)
    d_col = d_row.reshape(n, 1)

    contract_rows = (((0,), (0,)), ((), ()))
    adj16 = adj_vmem[...].astype(jnp.bfloat16)
    h = x
    for _ in range(2):
        y = (d_col * h).astype(jnp.bfloat16)
        t = lax.dot_general(adj16, y, contract_rows,
                            preferred_element_type=_F32)
        h = d_col * t

    h = jnp.maximum(
        jnp.dot(h, sgw_ref[...], preferred_element_type=_F32)
        + sgb_ref[...], 0.0)
    x_graph = jnp.maximum(
        jnp.dot(h, gdw_ref[...], preferred_element_type=_F32)
        + gdb_ref[...], 0.0)

    f_graph = x_graph.shape[1]
    p = jnp.maximum(
        jnp.dot(x_graph, p1w_ref[:f_graph, :], preferred_element_type=_F32)
        + jnp.dot(x, p1w_ref[f_graph:, :], preferred_element_type=_F32)
        + p1b_ref[...], 0.0)
    p = jnp.maximum(
        jnp.dot(p, p2w_ref[...], preferred_element_type=_F32)
        + p2b_ref[...], 0.0)
    value = jnp.dot(p, vw_ref[...], preferred_element_type=_F32) + vb_ref[...]
    out_ref[...] = value * mask_ref[...]


def kernel(features, adjacency, mask, enc1_w, enc1_b, enc2_w, enc2_b,
           sg_w, sg_b, gd_w, gd_b, p1_w, p1_b, p2_w, p2_b, v_w, v_b):
    n = features.shape[0]
    args = (
        features, adjacency, mask.reshape(n, 1),
        enc1_w, enc1_b.reshape(1, -1), enc2_w, enc2_b.reshape(1, -1),
        sg_w, sg_b.reshape(1, -1), gd_w, gd_b.reshape(1, -1),
        p1_w, p1_b.reshape(1, -1),
        p2_w, p2_b.reshape(1, -1), v_w, v_b.reshape(1, -1),
    )
    in_specs = [pl.BlockSpec(memory_space=pl.ANY) if i == 1
                else pl.BlockSpec(memory_space=pltpu.MemorySpace.VMEM)
                for i in range(len(args))]
    return pl.pallas_call(
        _fused_kernel,
        out_shape=jax.ShapeDtypeStruct((n, 1), jnp.float32),
        in_specs=in_specs,
        scratch_shapes=[
            pltpu.VMEM((n, n), _F32),
            pltpu.SemaphoreType.DMA((_N_CHUNKS,)),
        ],
    )(*args)


# speedup vs baseline: 1.1464x; 1.1464x over previous
import jax
import jax.numpy as jnp
from jax import lax
from jax.experimental import pallas as pl
from jax.experimental.pallas import tpu as pltpu

_F32 = jnp.float32
_N_CHUNKS = 4


def _row(ref):
    return ref[...].reshape(1, -1)


def _fused_kernel(feat_ref, adj_hbm_ref, mask_ref,
                  e1w_ref, e1b_ref, e2w_ref, e2b_ref,
                  sgw_ref, sgb_ref, gdw_ref, gdb_ref,
                  p1w_ref, p1b_ref, p2w_ref, p2b_ref,
                  vw_ref, vb_ref, out_ref, adj_vmem, sems):
    n = adj_hbm_ref.shape[0]
    chunk = n // _N_CHUNKS

    copies = []
    for k in range(_N_CHUNKS):
        rows = pl.ds(k * chunk, chunk)
        cp = pltpu.make_async_copy(adj_hbm_ref.at[rows, :],
                                   adj_vmem.at[rows, :], sems.at[k])
        cp.start()
        copies.append(cp)

    x = jnp.maximum(
        jnp.dot(feat_ref[...], e1w_ref[...], preferred_element_type=_F32)
        + _row(e1b_ref), 0.0)
    x = jnp.maximum(
        jnp.dot(x, e2w_ref[...], preferred_element_type=_F32)
        + _row(e2b_ref), 0.0)

    deg = jnp.zeros((1, n), dtype=_F32)
    for k in range(_N_CHUNKS):
        copies[k].wait()
        deg = deg + jnp.sum(adj_vmem[pl.ds(k * chunk, chunk), :], axis=0,
                            keepdims=True)
    d_row = jnp.where(deg > 0.0, lax.rsqrt(deg), 0.0)
    d_col = d_row.reshape(n, 1)

    contract_rows = (((0,), (0,)), ((), ()))
    adj16 = adj_vmem[...].astype(jnp.bfloat16)
    h = x
    for _ in range(2):
        y = (d_col * h).astype(jnp.bfloat16)
        t = lax.dot_general(adj16, y, contract_rows,
                            preferred_element_type=_F32)
        h = d_col * t

    h = jnp.maximum(
        jnp.dot(h, sgw_ref[...], preferred_element_type=_F32)
        + _row(sgb_ref), 0.0)
    x_graph = jnp.maximum(
        jnp.dot(h, gdw_ref[...], preferred_element_type=_F32)
        + _row(gdb_ref), 0.0)

    f_graph = x_graph.shape[1]
    p = jnp.maximum(
        jnp.dot(x_graph, p1w_ref[:f_graph, :], preferred_element_type=_F32)
        + jnp.dot(x, p1w_ref[f_graph:, :], preferred_element_type=_F32)
        + _row(p1b_ref), 0.0)
    p = jnp.maximum(
        jnp.dot(p, p2w_ref[...], preferred_element_type=_F32)
        + _row(p2b_ref), 0.0)
    value = jnp.dot(p, vw_ref[...], preferred_element_type=_F32) + _row(vb_ref)
    out_ref[...] = value * mask_ref[...].reshape(n, 1)


def kernel(features, adjacency, mask, enc1_w, enc1_b, enc2_w, enc2_b,
           sg_w, sg_b, gd_w, gd_b, p1_w, p1_b, p2_w, p2_b, v_w, v_b):
    n = features.shape[0]
    args = (features, adjacency, mask,
            enc1_w, enc1_b, enc2_w, enc2_b,
            sg_w, sg_b, gd_w, gd_b,
            p1_w, p1_b, p2_w, p2_b, v_w, v_b)
    in_specs = [pl.BlockSpec(memory_space=pl.ANY) if i == 1
                else pl.BlockSpec(memory_space=pltpu.MemorySpace.VMEM)
                for i in range(len(args))]
    return pl.pallas_call(
        _fused_kernel,
        out_shape=jax.ShapeDtypeStruct((n, 1), jnp.float32),
        in_specs=in_specs,
        scratch_shapes=[
            pltpu.VMEM((n, n), _F32),
            pltpu.SemaphoreType.DMA((_N_CHUNKS,)),
        ],
    )(*args)


# speedup vs baseline: 1.2048x; 1.0509x over previous
import jax
import jax.numpy as jnp
from jax import lax
from jax.experimental import pallas as pl
from jax.experimental.pallas import tpu as pltpu

_F32 = jnp.float32
_N_CHUNKS = 8
_N_SMALL = 16


def _row(v):
    return v.reshape(1, -1)


def _fused_kernel(feat_hbm, adj_hbm, mask_hbm,
                  e1w_hbm, e1b_hbm, e2w_hbm, e2b_hbm,
                  sgw_hbm, sgb_hbm, gdw_hbm, gdb_hbm,
                  p1w_hbm, p1b_hbm, p2w_hbm, p2b_hbm,
                  vw_hbm, vb_hbm, out_ref,
                  feat_v, mask_v, e1w_v, e1b_v, e2w_v, e2b_v,
                  sgw_v, sgb_v, gdw_v, gdb_v,
                  p1w_v, p1b_v, p2w_v, p2b_v, vw_v, vb_v,
                  adj_v, adj16_v, s_sems, a_sems):
    n = adj_hbm.shape[0]
    chunk = n // _N_CHUNKS

    small = [(feat_hbm, feat_v), (mask_hbm, mask_v),
             (e1w_hbm, e1w_v), (e1b_hbm, e1b_v),
             (e2w_hbm, e2w_v), (e2b_hbm, e2b_v),
             (sgw_hbm, sgw_v), (sgb_hbm, sgb_v),
             (gdw_hbm, gdw_v), (gdb_hbm, gdb_v),
             (p1w_hbm, p1w_v), (p1b_hbm, p1b_v),
             (p2w_hbm, p2w_v), (p2b_hbm, p2b_v),
             (vw_hbm, vw_v), (vb_hbm, vb_v)]
    scopies = []
    for i, (src, dst) in enumerate(small):
        cp = pltpu.make_async_copy(src, dst, s_sems.at[i])
        cp.start()
        scopies.append(cp)
    acopies = []
    for k in range(_N_CHUNKS):
        rows = pl.ds(k * chunk, chunk)
        cp = pltpu.make_async_copy(adj_hbm.at[rows, :], adj_v.at[rows, :],
                                   a_sems.at[k])
        cp.start()
        acopies.append(cp)

    for i in range(6):
        scopies[i].wait()
    x = jnp.maximum(
        jnp.dot(feat_v[...], e1w_v[...], preferred_element_type=_F32)
        + _row(e1b_v[...]), 0.0)
    x = jnp.maximum(
        jnp.dot(x, e2w_v[...], preferred_element_type=_F32)
        + _row(e2b_v[...]), 0.0)

    deg = jnp.zeros((1, n), dtype=_F32)
    for k in range(_N_CHUNKS):
        acopies[k].wait()
        rows = pl.ds(k * chunk, chunk)
        blk = adj_v[rows, :]
        deg = deg + jnp.sum(blk, axis=0, keepdims=True)
        adj16_v[rows, :] = blk.astype(jnp.bfloat16)
    d_row = jnp.where(deg > 0.0, lax.rsqrt(deg), 0.0)
    d_col = d_row.reshape(n, 1)

    contract_rows = (((0,), (0,)), ((), ()))
    adj16 = adj16_v[...]
    h = x
    for _ in range(2):
        y = (d_col * h).astype(jnp.bfloat16)
        t = lax.dot_general(adj16, y, contract_rows,
                            preferred_element_type=_F32)
        h = d_col * t

    for i in range(6, _N_SMALL):
        scopies[i].wait()
    h = jnp.maximum(
        jnp.dot(h, sgw_v[...], preferred_element_type=_F32)
        + _row(sgb_v[...]), 0.0)
    x_graph = jnp.maximum(
        jnp.dot(h, gdw_v[...], preferred_element_type=_F32)
        + _row(gdb_v[...]), 0.0)

    f_graph = x_graph.shape[1]
    p = jnp.maximum(
        jnp.dot(x_graph, p1w_v[:f_graph, :], preferred_element_type=_F32)
        + jnp.dot(x, p1w_v[f_graph:, :], preferred_element_type=_F32)
        + _row(p1b_v[...]), 0.0)
    p = jnp.maximum(
        jnp.dot(p, p2w_v[...], preferred_element_type=_F32)
        + _row(p2b_v[...]), 0.0)
    value = (jnp.dot(p, vw_v[...], preferred_element_type=_F32)
             + _row(vb_v[...]))
    out_ref[...] = value * mask_v[...].reshape(n, 1)


def kernel(features, adjacency, mask, enc1_w, enc1_b, enc2_w, enc2_b,
           sg_w, sg_b, gd_w, gd_b, p1_w, p1_b, p2_w, p2_b, v_w, v_b):
    n = features.shape[0]
    args = (features, adjacency, mask,
            enc1_w, enc1_b, enc2_w, enc2_b,
            sg_w, sg_b, gd_w, gd_b,
            p1_w, p1_b, p2_w, p2_b, v_w, v_b)
    vmem_of = lambda a: pltpu.VMEM(a.shape, a.dtype)
    small_scratch = [vmem_of(a) for a in
                     (features, mask, enc1_w, enc1_b, enc2_w, enc2_b,
                      sg_w, sg_b, gd_w, gd_b,
                      p1_w, p1_b, p2_w, p2_b, v_w, v_b)]
    return pl.pallas_call(
        _fused_kernel,
        out_shape=jax.ShapeDtypeStruct((n, 1), jnp.float32),
        in_specs=[pl.BlockSpec(memory_space=pl.ANY)] * len(args),
        scratch_shapes=small_scratch + [
            pltpu.VMEM((n, n), _F32),
            pltpu.VMEM((n, n), jnp.bfloat16),
            pltpu.SemaphoreType.DMA((_N_SMALL,)),
            pltpu.SemaphoreType.DMA((_N_CHUNKS,)),
        ],
    )(*args)
